# baseline (device time: 73182 ns/iter reference)
import os
import numpy as np
import jax
import jax.numpy as jnp
from jax import lax
from jax.experimental import pallas as pl
from jax.experimental.pallas import tpu as pltpu

N_DEV = 16
SQ = 1024
D = 1024
HQ = 8
DH = 128
HD = HQ * DH
HALF = SQ // 2
COLS = D // 2
SCALE = 0.08838834764831843
SIZES = (512, 256, 128, 64)


def _rope_tables():
    inv = 1.0 / (10000.0 ** (np.arange(0, DH, 2) / DH))
    pos = np.arange(SQ)[:, None] * inv[None, :]
    cos = np.repeat(np.cos(pos), 2, axis=-1).astype(np.float32)
    sin = np.repeat(np.sin(pos), 2, axis=-1).astype(np.float32)
    return cos, sin


_COS, _SIN = _rope_tables()
_ABLATE = os.environ.get("KERNEL_ABLATE", "")


def kernel(x, Wq, Wk, Wv, Wo):
    x2 = x.reshape(SQ, D)
    cos = jnp.asarray(_COS)
    sin = jnp.asarray(_SIN)

    def body(x_ref, wq_ref, wk_ref, wv_ref, wo_ref, cos_ref, sin_ref,
             out_ref, outb_ref, q_ref, k_ref, v_ref, ctx_ref, part_ref,
             sA0, sA1, sA2, sA3, sB0, sB1, sB2, sB3,
             rA0, rA1, rA2, rA3, rB0, rB1, rB2, rB3,
             rs_send, rs_recv, ag_send, ag_recv):
        my = lax.axis_index("i")
        z = my // 4
        r = my % 4
        y = jnp.where(r >= 2, 1, 0)
        xc = jnp.where((r == 1) | (r == 2), 1, 0)
        z0 = z % 2
        z1 = z // 2

        def lid(xx, yy, zz):
            return 4 * zz + 3 * yy + xx * (1 - 2 * yy)

        p_y = lid(xc, 1 - y, z)
        p_x = lid(1 - xc, y, z)
        p_z0 = lid(xc, y, z + 1 - 2 * z0)
        p_z1 = lid(xc, y, z + 2 - 4 * z1)

        def offsets(bits):
            keep, send = [], []
            base = 0
            for i, b in enumerate(bits):
                sz = SIZES[i]
                keep.append(base + b * sz)
                send.append(base + (1 - b) * sz)
                base = keep[i]
            return keep, send

        keepA, sendA = offsets([y, xc, z0, z1])
        keepB, sendB = offsets([z0, y, xc, z1])
        trees = [
            dict(ti=0, c0=0, partners=[p_y, p_x, p_z0, p_z1],
                 keep=keepA, send=sendA,
                 sbufs=[sA0, sA1, sA2, sA3], rbufs=[rA0, rA1, rA2, rA3]),
            dict(ti=1, c0=COLS, partners=[p_z0, p_y, p_x, p_z1],
                 keep=keepB, send=sendB,
                 sbufs=[sB0, sB1, sB2, sB3], rbufs=[rB0, rB1, rB2, rB3]),
        ]

        if _ABLATE != "nocomm":
            barrier = pltpu.get_barrier_semaphore()
            for p in (p_y, p_x, p_z0, p_z1):
                pl.semaphore_signal(barrier, inc=1, device_id=(p,),
                                    device_id_type=pl.DeviceIdType.MESH)
            pl.semaphore_wait(barrier, 4)

        xv = x_ref[:, :]
        cosv = jnp.concatenate([cos_ref[:, :]] * HQ, axis=1)
        sinv = jnp.concatenate([sin_ref[:, :]] * HQ, axis=1)
        col = lax.broadcasted_iota(jnp.int32, (SQ, HD), 1)
        even = (col % 2) == 0

        def rope(t):
            t_next = pltpu.roll(t, HD - 1, 1)
            t_prev = pltpu.roll(t, 1, 1)
            t_r = jnp.where(even, -t_next, t_prev)
            return t * cosv + t_r * sinv

        if _ABLATE != "nocompute":
            xvb = xv.astype(jnp.bfloat16)
            q_ref[:, :] = rope(jnp.dot(
                xvb, wq_ref[:, :].astype(jnp.bfloat16),
                preferred_element_type=jnp.float32)).astype(jnp.bfloat16)
            k_ref[:, :] = rope(jnp.dot(
                xvb, wk_ref[:, :].astype(jnp.bfloat16),
                preferred_element_type=jnp.float32)).astype(jnp.bfloat16)
            v_ref[:, :] = jnp.dot(
                xvb, wv_ref[:, :].astype(jnp.bfloat16),
                preferred_element_type=jnp.float32).astype(jnp.bfloat16)

        def attn_part(base, n):
            for h in range(HQ):
                sl = slice(h * DH, (h + 1) * DH)
                s = lax.dot_general(
                    q_ref[pl.ds(base, n), sl], k_ref[:, sl],
                    (((1,), (1,)), ((), ())),
                    preferred_element_type=jnp.float32) * SCALE
                w = jnp.exp(s)
                denom = jnp.sum(w, axis=1, keepdims=True)
                ctx = jnp.dot(w.astype(jnp.bfloat16), v_ref[:, sl],
                              preferred_element_type=jnp.float32)
                ctx_ref[0:n, sl] = (ctx / denom).astype(jnp.bfloat16)
            part_ref[pl.ds(base, n), :] = jnp.dot(
                ctx_ref[0:n, :], wo_ref[:, :].astype(jnp.bfloat16),
                preferred_element_type=jnp.float32)

        def rs_start(t, s):
            t["sbufs"][s][:, :] = part_ref[
                pl.ds(t["send"][s], SIZES[s]),
                t["c0"]:t["c0"] + COLS].astype(jnp.bfloat16)
            rdma = pltpu.make_async_remote_copy(
                src_ref=t["sbufs"][s],
                dst_ref=t["rbufs"][s],
                send_sem=rs_send.at[t["ti"], s],
                recv_sem=rs_recv.at[t["ti"], s],
                device_id=(t["partners"][s],),
                device_id_type=pl.DeviceIdType.MESH,
            )
            rdma.start()
            return rdma

        def acc(t, s_buf, rows0, nrows):
            roff = rows0 - t["keep"][s_buf]
            part_ref[pl.ds(rows0, nrows), t["c0"]:t["c0"] + COLS] = (
                part_ref[pl.ds(rows0, nrows), t["c0"]:t["c0"] + COLS]
                + t["rbufs"][s_buf][pl.ds(roff, nrows), :].astype(jnp.float32))

        _E = {3: 0, 2: 1, 1: 3, 0: 6}
        _F = {(3, 2): 2, (3, 1): 4, (2, 1): 5,
              (3, 0): 7, (2, 0): 8, (1, 0): 9}

        def ag_send_region(t, idx, rows0, nrows, to_s):
            rdma = pltpu.make_async_remote_copy(
                src_ref=outb_ref.at[pl.ds(rows0, nrows), pl.ds(t["c0"], COLS)],
                dst_ref=outb_ref.at[pl.ds(rows0, nrows), pl.ds(t["c0"], COLS)],
                send_sem=ag_send.at[t["ti"], idx],
                recv_sem=ag_recv.at[t["ti"], idx],
                device_id=(t["partners"][to_s],),
                device_id_type=pl.DeviceIdType.MESH,
            )
            rdma.start()
            return rdma

        A, B = trees

        if _ABLATE == "nocompute":
            part_ref[:, :] = x_ref[:, :]
        elif _ABLATE == "nocomm":
            attn_part(sendA[0], HALF)
            attn_part(keepA[0], HALF)
            out_ref[:, :] = part_ref[:, :]
            return

        def rs0_sub(t, sub, sem_idx):
            off = sub * (SIZES[0] // 2)
            t["sbufs"][0][pl.ds(off, SIZES[0] // 2), :] = part_ref[
                pl.ds(t["send"][0] + off, SIZES[0] // 2),
                t["c0"]:t["c0"] + COLS].astype(jnp.bfloat16)
            rdma = pltpu.make_async_remote_copy(
                src_ref=t["sbufs"][0].at[pl.ds(off, SIZES[0] // 2), :],
                dst_ref=t["rbufs"][0].at[pl.ds(off, SIZES[0] // 2), :],
                send_sem=rs_send.at[t["ti"], sem_idx],
                recv_sem=rs_recv.at[t["ti"], sem_idx],
                device_id=(t["partners"][0],),
                device_id_type=pl.DeviceIdType.MESH,
            )
            rdma.start()
            return rdma

        if _ABLATE != "nocompute":
            attn_part(sendA[0], HALF // 2)
            a1 = rs0_sub(A, 0, 4)
            attn_part(sendA[0] + HALF // 2, HALF // 2)
            a2 = rs0_sub(A, 1, 0)
            attn_part(keepA[0], HALF)

            class _Both:
                def wait(self):
                    a1.wait()
                    a2.wait()

            a = _Both()
        else:
            a = rs_start(A, 0)
        b = rs_start(B, 0)
        for s in (1, 2, 3):
            a.wait()
            acc(A, s - 1, A["send"][s], SIZES[s])
            a = rs_start(A, s)
            acc(A, s - 1, A["keep"][s], SIZES[s])
            b.wait()
            acc(B, s - 1, B["send"][s], SIZES[s])
            b = rs_start(B, s)
            acc(B, s - 1, B["keep"][s], SIZES[s])
        dA, dB = {}, {}
        a.wait()
        acc(A, 3, A["keep"][3], 64)
        outb_ref[pl.ds(keepA[3], 64), pl.ds(0, COLS)] = part_ref[
            pl.ds(keepA[3], 64), 0:COLS].astype(jnp.bfloat16)
        for s in (3, 2, 1, 0):
            dA[_E[s]] = ag_send_region(A, _E[s], A["keep"][3], 64, s)
        b.wait()
        acc(B, 3, B["keep"][3], 64)
        outb_ref[pl.ds(keepB[3], 64), pl.ds(COLS, COLS)] = part_ref[
            pl.ds(keepB[3], 64), COLS:D].astype(jnp.bfloat16)
        for s in (3, 2, 1, 0):
            dB[_E[s]] = ag_send_region(B, _E[s], B["keep"][3], 64, s)

        def ag_stage1(t, d):
            d[_E[3]].wait()
            for to in (2, 1, 0):
                d[_F[(3, to)]] = ag_send_region(
                    t, _F[(3, to)], t["send"][3], 64, to)

        def ag_stage2(t, d):
            d[_E[2]].wait()
            d[_F[(3, 2)]].wait()
            for to in (1, 0):
                d[_F[(2, to)]] = ag_send_region(
                    t, _F[(2, to)], t["send"][2], 128, to)

        def ag_stage3(t, d):
            d[_E[1]].wait()
            d[_F[(3, 1)]].wait()
            d[_F[(2, 1)]].wait()
            d[_F[(1, 0)]] = ag_send_region(
                t, _F[(1, 0)], t["send"][1], 256, 0)

        def ag_stage4(t, d):
            d[_E[0]].wait()
            d[_F[(3, 0)]].wait()
            d[_F[(2, 0)]].wait()
            d[_F[(1, 0)]].wait()

        for stage in (ag_stage1, ag_stage2, ag_stage3, ag_stage4):
            stage(A, dA)
            stage(B, dB)

        out_ref[:, :] = outb_ref[:, :].astype(jnp.float32)

    out = pl.pallas_call(
        body,
        out_shape=jax.ShapeDtypeStruct((SQ, D), jnp.float32),
        in_specs=[pl.BlockSpec(memory_space=pltpu.VMEM)] * 7,
        out_specs=pl.BlockSpec(memory_space=pltpu.VMEM),
        scratch_shapes=[
            pltpu.VMEM((SQ, D), jnp.bfloat16),
            pltpu.VMEM((SQ, HD), jnp.bfloat16),
            pltpu.VMEM((SQ, HD), jnp.bfloat16),
            pltpu.VMEM((SQ, HD), jnp.bfloat16),
            pltpu.VMEM((HALF, HD), jnp.bfloat16),
            pltpu.VMEM((SQ, D), jnp.float32),
            pltpu.VMEM((512, COLS), jnp.bfloat16),
            pltpu.VMEM((256, COLS), jnp.bfloat16),
            pltpu.VMEM((128, COLS), jnp.bfloat16),
            pltpu.VMEM((64, COLS), jnp.bfloat16),
            pltpu.VMEM((512, COLS), jnp.bfloat16),
            pltpu.VMEM((256, COLS), jnp.bfloat16),
            pltpu.VMEM((128, COLS), jnp.bfloat16),
            pltpu.VMEM((64, COLS), jnp.bfloat16),
            pltpu.VMEM((512, COLS), jnp.bfloat16),
            pltpu.VMEM((256, COLS), jnp.bfloat16),
            pltpu.VMEM((128, COLS), jnp.bfloat16),
            pltpu.VMEM((64, COLS), jnp.bfloat16),
            pltpu.VMEM((512, COLS), jnp.bfloat16),
            pltpu.VMEM((256, COLS), jnp.bfloat16),
            pltpu.VMEM((128, COLS), jnp.bfloat16),
            pltpu.VMEM((64, COLS), jnp.bfloat16),
            pltpu.SemaphoreType.DMA((2, 5)),
            pltpu.SemaphoreType.DMA((2, 5)),
            pltpu.SemaphoreType.DMA((2, 10)),
            pltpu.SemaphoreType.DMA((2, 10)),
        ],
        compiler_params=pltpu.CompilerParams(
            collective_id=None if _ABLATE == "nocomm" else 0,
            vmem_limit_bytes=128 * 1024 * 1024,
        ),
    )(x2, Wq, Wk, Wv, Wo, cos, sin)
    return out.reshape(1, SQ, D)


# device time: 72261 ns/iter; 1.0127x vs baseline; 1.0127x over previous
import os
import numpy as np
import jax
import jax.numpy as jnp
from jax import lax
from jax.experimental import pallas as pl
from jax.experimental.pallas import tpu as pltpu

N_DEV = 16
SQ = 1024
D = 1024
HQ = 8
DH = 128
HD = HQ * DH
HALF = SQ // 2
COLS = D // 2
SCALE = 0.08838834764831843
SIZES = (512, 256, 128, 64)


def _rope_tables():
    inv = 1.0 / (10000.0 ** (np.arange(0, DH, 2) / DH))
    pos = np.arange(SQ)[:, None] * inv[None, :]
    cos = np.repeat(np.cos(pos), 2, axis=-1).astype(np.float32)
    sin = np.repeat(np.sin(pos), 2, axis=-1).astype(np.float32)
    return cos, sin


_COS, _SIN = _rope_tables()
_ABLATE = os.environ.get("KERNEL_ABLATE", "")


def kernel(x, Wq, Wk, Wv, Wo):
    x2 = x.reshape(SQ, D)
    cos = jnp.asarray(_COS)
    sin = jnp.asarray(_SIN)

    def body(x_ref, wq_ref, wk_ref, wv_ref, wo_ref, cos_ref, sin_ref,
             out_ref, outb_ref, q_ref, k_ref, v_ref, ctx_ref, part_ref,
             sA0, sA1, sA2, sA3, sB0, sB1, sB2, sB3,
             rA0, rA1, rA2, rA3, rB0, rB1, rB2, rB3,
             rs_send, rs_recv, ag_send, ag_recv):
        my = lax.axis_index("i")
        z = my // 4
        r = my % 4
        y = jnp.where(r >= 2, 1, 0)
        xc = jnp.where((r == 1) | (r == 2), 1, 0)
        z0 = z % 2
        z1 = z // 2

        def lid(xx, yy, zz):
            return 4 * zz + 3 * yy + xx * (1 - 2 * yy)

        p_y = lid(xc, 1 - y, z)
        p_x = lid(1 - xc, y, z)
        p_z0 = lid(xc, y, z + 1 - 2 * z0)
        p_z1 = lid(xc, y, z + 2 - 4 * z1)

        def offsets(bits):
            keep, send = [], []
            base = 0
            for i, b in enumerate(bits):
                sz = SIZES[i]
                keep.append(base + b * sz)
                send.append(base + (1 - b) * sz)
                base = keep[i]
            return keep, send

        keepA, sendA = offsets([y, xc, z0, z1])
        keepB, sendB = offsets([z0, y, xc, z1])
        trees = [
            dict(ti=0, c0=0, partners=[p_y, p_x, p_z0, p_z1],
                 keep=keepA, send=sendA,
                 sbufs=[sA0, sA1, sA2, sA3], rbufs=[rA0, rA1, rA2, rA3]),
            dict(ti=1, c0=COLS, partners=[p_z0, p_y, p_x, p_z1],
                 keep=keepB, send=sendB,
                 sbufs=[sB0, sB1, sB2, sB3], rbufs=[rB0, rB1, rB2, rB3]),
        ]

        if _ABLATE != "nocomm":
            barrier = pltpu.get_barrier_semaphore()
            for p in (p_y, p_x, p_z0, p_z1):
                pl.semaphore_signal(barrier, inc=1, device_id=(p,),
                                    device_id_type=pl.DeviceIdType.MESH)
            pl.semaphore_wait(barrier, 4)

        xv = x_ref[:, :]
        cosv = jnp.concatenate([cos_ref[:, :]] * HQ, axis=1)
        sinv = jnp.concatenate([sin_ref[:, :]] * HQ, axis=1)
        col = lax.broadcasted_iota(jnp.int32, (SQ, HD), 1)
        even = (col % 2) == 0

        def rope(t):
            t_next = pltpu.roll(t, HD - 1, 1)
            t_prev = pltpu.roll(t, 1, 1)
            t_r = jnp.where(even, -t_next, t_prev)
            return t * cosv + t_r * sinv

        if _ABLATE != "nocompute":
            xvb = xv.astype(jnp.bfloat16)
            q_ref[:, :] = rope(jnp.dot(
                xvb, wq_ref[:, :].astype(jnp.bfloat16),
                preferred_element_type=jnp.float32)).astype(jnp.bfloat16)
            k_ref[:, :] = rope(jnp.dot(
                xvb, wk_ref[:, :].astype(jnp.bfloat16),
                preferred_element_type=jnp.float32)).astype(jnp.bfloat16)
            v_ref[:, :] = jnp.dot(
                xvb, wv_ref[:, :].astype(jnp.bfloat16),
                preferred_element_type=jnp.float32).astype(jnp.bfloat16)

        def attn_part(base, n):
            for h in range(HQ):
                sl = slice(h * DH, (h + 1) * DH)
                s = lax.dot_general(
                    q_ref[pl.ds(base, n), sl], k_ref[:, sl],
                    (((1,), (1,)), ((), ())),
                    preferred_element_type=jnp.float32) * SCALE
                w = jnp.exp(s)
                denom = jnp.sum(w, axis=1, keepdims=True)
                ctx = jnp.dot(w.astype(jnp.bfloat16), v_ref[:, sl],
                              preferred_element_type=jnp.float32)
                ctx_ref[0:n, sl] = (ctx / denom).astype(jnp.bfloat16)
            part_ref[pl.ds(base, n), :] = jnp.dot(
                ctx_ref[0:n, :], wo_ref[:, :].astype(jnp.bfloat16),
                preferred_element_type=jnp.float32)

        def rs_start(t, s):
            t["sbufs"][s][:, :] = part_ref[
                pl.ds(t["send"][s], SIZES[s]),
                t["c0"]:t["c0"] + COLS].astype(jnp.bfloat16)
            rdma = pltpu.make_async_remote_copy(
                src_ref=t["sbufs"][s],
                dst_ref=t["rbufs"][s],
                send_sem=rs_send.at[t["ti"], s],
                recv_sem=rs_recv.at[t["ti"], s],
                device_id=(t["partners"][s],),
                device_id_type=pl.DeviceIdType.MESH,
            )
            rdma.start()
            return rdma

        def acc(t, s_buf, rows0, nrows):
            roff = rows0 - t["keep"][s_buf]
            part_ref[pl.ds(rows0, nrows), t["c0"]:t["c0"] + COLS] = (
                part_ref[pl.ds(rows0, nrows), t["c0"]:t["c0"] + COLS]
                + t["rbufs"][s_buf][pl.ds(roff, nrows), :].astype(jnp.float32))

        _E = {3: 0, 2: 1, 1: 3, 0: 6}
        _F = {(3, 2): 2, (3, 1): 4, (2, 1): 5,
              (3, 0): 7, (2, 0): 8, (1, 0): 9}

        def ag_send_region(t, idx, rows0, nrows, to_s):
            rdma = pltpu.make_async_remote_copy(
                src_ref=outb_ref.at[pl.ds(rows0, nrows), pl.ds(t["c0"], COLS)],
                dst_ref=outb_ref.at[pl.ds(rows0, nrows), pl.ds(t["c0"], COLS)],
                send_sem=ag_send.at[t["ti"], idx],
                recv_sem=ag_recv.at[t["ti"], idx],
                device_id=(t["partners"][to_s],),
                device_id_type=pl.DeviceIdType.MESH,
            )
            rdma.start()
            return rdma

        A, B = trees

        if _ABLATE == "nocompute":
            part_ref[:, :] = x_ref[:, :]
        elif _ABLATE == "nocomm":
            attn_part(sendA[0], HALF)
            attn_part(keepA[0], HALF)
            out_ref[:, :] = part_ref[:, :]
            return

        def rs0_sub(t, sub, sem_idx):
            off = sub * (SIZES[0] // 2)
            t["sbufs"][0][pl.ds(off, SIZES[0] // 2), :] = part_ref[
                pl.ds(t["send"][0] + off, SIZES[0] // 2),
                t["c0"]:t["c0"] + COLS].astype(jnp.bfloat16)
            rdma = pltpu.make_async_remote_copy(
                src_ref=t["sbufs"][0].at[pl.ds(off, SIZES[0] // 2), :],
                dst_ref=t["rbufs"][0].at[pl.ds(off, SIZES[0] // 2), :],
                send_sem=rs_send.at[t["ti"], sem_idx],
                recv_sem=rs_recv.at[t["ti"], sem_idx],
                device_id=(t["partners"][0],),
                device_id_type=pl.DeviceIdType.MESH,
            )
            rdma.start()
            return rdma

        if _ABLATE != "nocompute":
            attn_part(sendA[0], HALF)
        a = rs_start(A, 0)
        if _ABLATE != "nocompute":
            attn_part(keepA[0], HALF)
        b = rs_start(B, 0)
        for s in (1, 2, 3):
            a.wait()
            acc(A, s - 1, A["send"][s], SIZES[s])
            a = rs_start(A, s)
            acc(A, s - 1, A["keep"][s], SIZES[s])
            b.wait()
            acc(B, s - 1, B["send"][s], SIZES[s])
            b = rs_start(B, s)
            acc(B, s - 1, B["keep"][s], SIZES[s])
        dA, dB = {}, {}
        a.wait()
        acc(A, 3, A["keep"][3], 64)
        outb_ref[pl.ds(keepA[3], 64), pl.ds(0, COLS)] = part_ref[
            pl.ds(keepA[3], 64), 0:COLS].astype(jnp.bfloat16)
        for s in (3, 2, 1, 0):
            dA[_E[s]] = ag_send_region(A, _E[s], A["keep"][3], 64, s)
        b.wait()
        acc(B, 3, B["keep"][3], 64)
        outb_ref[pl.ds(keepB[3], 64), pl.ds(COLS, COLS)] = part_ref[
            pl.ds(keepB[3], 64), COLS:D].astype(jnp.bfloat16)
        for s in (3, 2, 1, 0):
            dB[_E[s]] = ag_send_region(B, _E[s], B["keep"][3], 64, s)

        def ag_stage1(t, d):
            d[_E[3]].wait()
            for to in (2, 1, 0):
                d[_F[(3, to)]] = ag_send_region(
                    t, _F[(3, to)], t["send"][3], 64, to)

        def ag_stage2(t, d):
            d[_E[2]].wait()
            d[_F[(3, 2)]].wait()
            for to in (1, 0):
                d[_F[(2, to)]] = ag_send_region(
                    t, _F[(2, to)], t["send"][2], 128, to)

        def ag_stage3(t, d):
            d[_E[1]].wait()
            d[_F[(3, 1)]].wait()
            d[_F[(2, 1)]].wait()
            d[_F[(1, 0)]] = ag_send_region(
                t, _F[(1, 0)], t["send"][1], 256, 0)

        def ag_stage4(t, d):
            d[_E[0]].wait()
            d[_F[(3, 0)]].wait()
            d[_F[(2, 0)]].wait()
            d[_F[(1, 0)]].wait()

        for stage in (ag_stage1, ag_stage2, ag_stage3, ag_stage4):
            stage(A, dA)
            stage(B, dB)

        out_ref[:, :] = outb_ref[:, :].astype(jnp.float32)

    out = pl.pallas_call(
        body,
        out_shape=jax.ShapeDtypeStruct((SQ, D), jnp.float32),
        in_specs=[pl.BlockSpec(memory_space=pltpu.VMEM)] * 7,
        out_specs=pl.BlockSpec(memory_space=pltpu.VMEM),
        scratch_shapes=[
            pltpu.VMEM((SQ, D), jnp.bfloat16),
            pltpu.VMEM((SQ, HD), jnp.bfloat16),
            pltpu.VMEM((SQ, HD), jnp.bfloat16),
            pltpu.VMEM((SQ, HD), jnp.bfloat16),
            pltpu.VMEM((HALF, HD), jnp.bfloat16),
            pltpu.VMEM((SQ, D), jnp.float32),
            pltpu.VMEM((512, COLS), jnp.bfloat16),
            pltpu.VMEM((256, COLS), jnp.bfloat16),
            pltpu.VMEM((128, COLS), jnp.bfloat16),
            pltpu.VMEM((64, COLS), jnp.bfloat16),
            pltpu.VMEM((512, COLS), jnp.bfloat16),
            pltpu.VMEM((256, COLS), jnp.bfloat16),
            pltpu.VMEM((128, COLS), jnp.bfloat16),
            pltpu.VMEM((64, COLS), jnp.bfloat16),
            pltpu.VMEM((512, COLS), jnp.bfloat16),
            pltpu.VMEM((256, COLS), jnp.bfloat16),
            pltpu.VMEM((128, COLS), jnp.bfloat16),
            pltpu.VMEM((64, COLS), jnp.bfloat16),
            pltpu.VMEM((512, COLS), jnp.bfloat16),
            pltpu.VMEM((256, COLS), jnp.bfloat16),
            pltpu.VMEM((128, COLS), jnp.bfloat16),
            pltpu.VMEM((64, COLS), jnp.bfloat16),
            pltpu.SemaphoreType.DMA((2, 5)),
            pltpu.SemaphoreType.DMA((2, 5)),
            pltpu.SemaphoreType.DMA((2, 10)),
            pltpu.SemaphoreType.DMA((2, 10)),
        ],
        compiler_params=pltpu.CompilerParams(
            collective_id=None if _ABLATE == "nocomm" else 0,
            vmem_limit_bytes=128 * 1024 * 1024,
        ),
    )(x2, Wq, Wk, Wv, Wo, cos, sin)
    return out.reshape(1, SQ, D)


# device time: 70744 ns/iter; 1.0345x vs baseline; 1.0214x over previous
import os
import numpy as np
import jax
import jax.numpy as jnp
from jax import lax
from jax.experimental import pallas as pl
from jax.experimental.pallas import tpu as pltpu

N_DEV = 16
SQ = 1024
D = 1024
HQ = 8
DH = 128
HD = HQ * DH
HALF = SQ // 2
COLS = D // 2
SCALE = 0.08838834764831843
SIZES = (512, 256, 128, 64)


def _rope_tables():
    inv = 1.0 / (10000.0 ** (np.arange(0, DH, 2) / DH))
    pos = np.arange(SQ)[:, None] * inv[None, :]
    cos = np.repeat(np.cos(pos), 2, axis=-1).astype(np.float32)
    sin = np.repeat(np.sin(pos), 2, axis=-1).astype(np.float32)
    return cos, sin


_COS, _SIN = _rope_tables()
_ABLATE = os.environ.get("KERNEL_ABLATE", "")


def kernel(x, Wq, Wk, Wv, Wo):
    x2 = x.reshape(SQ, D)
    cos = jnp.asarray(_COS)
    sin = jnp.asarray(_SIN)

    def body(x_ref, wq_ref, wk_ref, wv_ref, wo_ref, cos_ref, sin_ref,
             out_ref, q_ref, k_ref, v_ref, ctx_ref, part_ref,
             sA0, sA1, sA2, sA3, sB0, sB1, sB2, sB3,
             rA0, rA1, rA2, rA3, rB0, rB1, rB2, rB3,
             rs_send, rs_recv, ag_send, ag_recv):
        my = lax.axis_index("i")
        z = my // 4
        r = my % 4
        y = jnp.where(r >= 2, 1, 0)
        xc = jnp.where((r == 1) | (r == 2), 1, 0)
        z0 = z % 2
        z1 = z // 2

        def lid(xx, yy, zz):
            return 4 * zz + 3 * yy + xx * (1 - 2 * yy)

        p_y = lid(xc, 1 - y, z)
        p_x = lid(1 - xc, y, z)
        p_z0 = lid(xc, y, z + 1 - 2 * z0)
        p_z1 = lid(xc, y, z + 2 - 4 * z1)

        def offsets(bits):
            keep, send = [], []
            base = 0
            for i, b in enumerate(bits):
                sz = SIZES[i]
                keep.append(base + b * sz)
                send.append(base + (1 - b) * sz)
                base = keep[i]
            return keep, send

        keepA, sendA = offsets([y, xc, z0, z1])
        keepB, sendB = offsets([z0, y, xc, z1])
        trees = [
            dict(ti=0, c0=0, partners=[p_y, p_x, p_z0, p_z1],
                 keep=keepA, send=sendA,
                 sbufs=[sA0, sA1, sA2, sA3], rbufs=[rA0, rA1, rA2, rA3]),
            dict(ti=1, c0=COLS, partners=[p_z0, p_y, p_x, p_z1],
                 keep=keepB, send=sendB,
                 sbufs=[sB0, sB1, sB2, sB3], rbufs=[rB0, rB1, rB2, rB3]),
        ]

        if _ABLATE != "nocomm":
            barrier = pltpu.get_barrier_semaphore()
            for p in (p_y, p_x, p_z0, p_z1):
                pl.semaphore_signal(barrier, inc=1, device_id=(p,),
                                    device_id_type=pl.DeviceIdType.MESH)
            pl.semaphore_wait(barrier, 4)

        xv = x_ref[:, :]
        cosv = jnp.concatenate([cos_ref[:, :]] * HQ, axis=1)
        sinv = jnp.concatenate([sin_ref[:, :]] * HQ, axis=1)
        col = lax.broadcasted_iota(jnp.int32, (SQ, HD), 1)
        even = (col % 2) == 0

        def rope(t):
            t_next = pltpu.roll(t, HD - 1, 1)
            t_prev = pltpu.roll(t, 1, 1)
            t_r = jnp.where(even, -t_next, t_prev)
            return t * cosv + t_r * sinv

        if _ABLATE != "nocompute":
            xvb = xv.astype(jnp.bfloat16)
            q_ref[:, :] = rope(jnp.dot(
                xvb, wq_ref[:, :].astype(jnp.bfloat16),
                preferred_element_type=jnp.float32)).astype(jnp.bfloat16)
            k_ref[:, :] = rope(jnp.dot(
                xvb, wk_ref[:, :].astype(jnp.bfloat16),
                preferred_element_type=jnp.float32)).astype(jnp.bfloat16)
            v_ref[:, :] = jnp.dot(
                xvb, wv_ref[:, :].astype(jnp.bfloat16),
                preferred_element_type=jnp.float32).astype(jnp.bfloat16)

        def attn_part(base, n):
            for h in range(HQ):
                sl = slice(h * DH, (h + 1) * DH)
                s = lax.dot_general(
                    q_ref[pl.ds(base, n), sl], k_ref[:, sl],
                    (((1,), (1,)), ((), ())),
                    preferred_element_type=jnp.float32) * SCALE
                w = jnp.exp(s)
                denom = jnp.sum(w, axis=1, keepdims=True)
                ctx = jnp.dot(w.astype(jnp.bfloat16), v_ref[:, sl],
                              preferred_element_type=jnp.float32)
                ctx_ref[0:n, sl] = (ctx / denom).astype(jnp.bfloat16)
            part_ref[pl.ds(base, n), :] = jnp.dot(
                ctx_ref[0:n, :], wo_ref[:, :].astype(jnp.bfloat16),
                preferred_element_type=jnp.float32)

        def rs_start(t, s):
            t["sbufs"][s][:, :] = part_ref[
                pl.ds(t["send"][s], SIZES[s]),
                t["c0"]:t["c0"] + COLS].astype(jnp.bfloat16)
            rdma = pltpu.make_async_remote_copy(
                src_ref=t["sbufs"][s],
                dst_ref=t["rbufs"][s],
                send_sem=rs_send.at[t["ti"], s],
                recv_sem=rs_recv.at[t["ti"], s],
                device_id=(t["partners"][s],),
                device_id_type=pl.DeviceIdType.MESH,
            )
            rdma.start()
            return rdma

        def acc(t, s_buf, rows0, nrows):
            roff = rows0 - t["keep"][s_buf]
            part_ref[pl.ds(rows0, nrows), t["c0"]:t["c0"] + COLS] = (
                part_ref[pl.ds(rows0, nrows), t["c0"]:t["c0"] + COLS]
                + t["rbufs"][s_buf][pl.ds(roff, nrows), :].astype(jnp.float32))

        _E = {3: 0, 2: 1, 1: 3, 0: 6}
        _F = {(3, 2): 2, (3, 1): 4, (2, 1): 5,
              (3, 0): 7, (2, 0): 8, (1, 0): 9}

        def ag_send_region(t, idx, rows0, nrows, to_s):
            rdma = pltpu.make_async_remote_copy(
                src_ref=out_ref.at[pl.ds(rows0, nrows), pl.ds(t["c0"], COLS)],
                dst_ref=out_ref.at[pl.ds(rows0, nrows), pl.ds(t["c0"], COLS)],
                send_sem=ag_send.at[t["ti"], idx],
                recv_sem=ag_recv.at[t["ti"], idx],
                device_id=(t["partners"][to_s],),
                device_id_type=pl.DeviceIdType.MESH,
            )
            rdma.start()
            return rdma

        A, B = trees

        if _ABLATE == "nocompute":
            part_ref[:, :] = x_ref[:, :]
        elif _ABLATE == "nocomm":
            attn_part(sendA[0], HALF)
            attn_part(keepA[0], HALF)
            out_ref[:, :] = part_ref[:, :].astype(jnp.bfloat16)
            return

        if _ABLATE != "nocompute":
            attn_part(sendA[0], HALF)
        a = rs_start(A, 0)
        if _ABLATE != "nocompute":
            attn_part(keepA[0], HALF)
        b = rs_start(B, 0)
        for s in (1, 2, 3):
            a.wait()
            acc(A, s - 1, A["send"][s], SIZES[s])
            a = rs_start(A, s)
            acc(A, s - 1, A["keep"][s], SIZES[s])
            b.wait()
            acc(B, s - 1, B["send"][s], SIZES[s])
            b = rs_start(B, s)
            acc(B, s - 1, B["keep"][s], SIZES[s])
        dA, dB = {}, {}
        a.wait()
        acc(A, 3, A["keep"][3], 64)
        out_ref[pl.ds(keepA[3], 64), pl.ds(0, COLS)] = part_ref[
            pl.ds(keepA[3], 64), 0:COLS].astype(jnp.bfloat16)
        for s in (3, 2, 1, 0):
            dA[_E[s]] = ag_send_region(A, _E[s], A["keep"][3], 64, s)
        b.wait()
        acc(B, 3, B["keep"][3], 64)
        out_ref[pl.ds(keepB[3], 64), pl.ds(COLS, COLS)] = part_ref[
            pl.ds(keepB[3], 64), COLS:D].astype(jnp.bfloat16)
        for s in (3, 2, 1, 0):
            dB[_E[s]] = ag_send_region(B, _E[s], B["keep"][3], 64, s)

        def ag_stage1(t, d):
            d[_E[3]].wait()
            for to in (2, 1, 0):
                d[_F[(3, to)]] = ag_send_region(
                    t, _F[(3, to)], t["send"][3], 64, to)

        def ag_stage2(t, d):
            d[_E[2]].wait()
            d[_F[(3, 2)]].wait()
            for to in (1, 0):
                d[_F[(2, to)]] = ag_send_region(
                    t, _F[(2, to)], t["send"][2], 128, to)

        def ag_stage3(t, d):
            d[_E[1]].wait()
            d[_F[(3, 1)]].wait()
            d[_F[(2, 1)]].wait()
            d[_F[(1, 0)]] = ag_send_region(
                t, _F[(1, 0)], t["send"][1], 256, 0)

        def ag_stage4(t, d):
            d[_E[0]].wait()
            d[_F[(3, 0)]].wait()
            d[_F[(2, 0)]].wait()
            d[_F[(1, 0)]].wait()

        for stage in (ag_stage1, ag_stage2, ag_stage3, ag_stage4):
            stage(A, dA)
            stage(B, dB)

    out = pl.pallas_call(
        body,
        out_shape=jax.ShapeDtypeStruct((SQ, D), jnp.bfloat16),
        in_specs=[pl.BlockSpec(memory_space=pltpu.VMEM)] * 7,
        out_specs=pl.BlockSpec(memory_space=pltpu.VMEM),
        scratch_shapes=[
            pltpu.VMEM((SQ, HD), jnp.bfloat16),
            pltpu.VMEM((SQ, HD), jnp.bfloat16),
            pltpu.VMEM((SQ, HD), jnp.bfloat16),
            pltpu.VMEM((HALF, HD), jnp.bfloat16),
            pltpu.VMEM((SQ, D), jnp.float32),
            pltpu.VMEM((512, COLS), jnp.bfloat16),
            pltpu.VMEM((256, COLS), jnp.bfloat16),
            pltpu.VMEM((128, COLS), jnp.bfloat16),
            pltpu.VMEM((64, COLS), jnp.bfloat16),
            pltpu.VMEM((512, COLS), jnp.bfloat16),
            pltpu.VMEM((256, COLS), jnp.bfloat16),
            pltpu.VMEM((128, COLS), jnp.bfloat16),
            pltpu.VMEM((64, COLS), jnp.bfloat16),
            pltpu.VMEM((512, COLS), jnp.bfloat16),
            pltpu.VMEM((256, COLS), jnp.bfloat16),
            pltpu.VMEM((128, COLS), jnp.bfloat16),
            pltpu.VMEM((64, COLS), jnp.bfloat16),
            pltpu.VMEM((512, COLS), jnp.bfloat16),
            pltpu.VMEM((256, COLS), jnp.bfloat16),
            pltpu.VMEM((128, COLS), jnp.bfloat16),
            pltpu.VMEM((64, COLS), jnp.bfloat16),
            pltpu.SemaphoreType.DMA((2, 5)),
            pltpu.SemaphoreType.DMA((2, 5)),
            pltpu.SemaphoreType.DMA((2, 10)),
            pltpu.SemaphoreType.DMA((2, 10)),
        ],
        compiler_params=pltpu.CompilerParams(
            collective_id=None if _ABLATE == "nocomm" else 0,
            vmem_limit_bytes=128 * 1024 * 1024,
        ),
    )(x2, Wq, Wk, Wv, Wo, cos, sin)
    return out.astype(jnp.float32).reshape(1, SQ, D)
